# Initial kernel scaffold; baseline (speedup 1.0000x reference)
import jax
import jax.numpy as jnp
from jax.experimental import pallas as pl


def kernel(observations, params):
    raise NotImplementedError("write your pallas kernel here")


# speedup vs baseline: 426.6957x; 426.6957x over previous
import functools

import jax
import jax.numpy as jnp
from jax import lax
from jax.experimental import pallas as pl
from jax.experimental.pallas import tpu as pltpu
from jax.experimental.pallas import tpu_sc as plsc

_B = 4
_OBS = 262144
_N = 256
_NC = 2
_NS = 16
_NW = _NC * _NS
_CHUNK = _OBS // _NS
_ROWS_W = _N // _NW


def _sc_extract_kernel(obs, mask_out, feat_out,
                       cntbuf, cntmat, shared, allcnt, sumbuf,
                       rowbuf, maskrow, posbuf, trbuf, featbuf):
    cid = lax.axis_index("c")
    sid = lax.axis_index("s")
    wid = sid * _NC + cid
    iota = lax.iota(jnp.int32, 16)
    zeros_i = jnp.zeros((16,), jnp.int32)
    ones_i = jnp.ones((16,), jnp.int32)
    zeros_f = jnp.zeros((16,), jnp.float32)

    for s in range(_B):
        pltpu.sync_copy(obs.at[pl.ds(s * _OBS + sid * _CHUNK, _CHUNK)],
                        cntbuf)

        def cbody(i, acc):
            for u in range(8):
                v = cntbuf[pl.ds(i * 128 + u * 16, 16)]
                acc = acc + jnp.where(v != 0.0, ones_i, zeros_i)
            return acc

        cntmat[s] = lax.fori_loop(0, _CHUNK // 128, cbody, zeros_i)

    pltpu.sync_copy(cntmat, shared.at[sid])
    plsc.subcore_barrier()
    pltpu.sync_copy(shared, allcnt)

    n_list = []
    for s in range(_B):
        tot = zeros_i
        for r in range(_NS):
            tot = tot + allcnt[r, s]
        nz = tot[0]
        for k in range(1, 16):
            nz = nz + tot[k]
        r0 = jnp.int32(0)
        for b in [512, 256, 128, 64, 32, 16, 8, 4, 2, 1]:
            t = r0 + b
            r0 = jnp.where(t * t <= nz, t, r0)
        n_list.append(jnp.minimum(jnp.maximum(r0 >> 1, 5), _N))

    for s in range(_B):
        n_s = n_list[s]
        n_f = n_s.astype(jnp.float32)

        def rowbody(k, carry):
            i = wid * _ROWS_W + k
            row_on = i < n_s

            mrow_off = pl.multiple_of(s * _N * _N + i * _N, 8)

            @pl.when(row_on)
            def _():
                start = i * n_s
                abase = pl.multiple_of((s * _OBS + start) & (-8), 8)
                off = (s * _OBS + start) - abase
                pltpu.sync_copy(obs.at[pl.ds(abase, 384)], rowbuf)
                for j in range(16):
                    col = j * 16 + iota
                    vals = plsc.load_gather(rowbuf, [off + col])
                    act = (vals != 0.0) & (col < n_s)
                    maskrow[pl.ds(j * 16, 16)] = jnp.where(act, 1.0, 0.0)
                pltpu.sync_copy(maskrow, mask_out.at[pl.ds(mrow_off, _N)])

            @pl.when(jnp.logical_not(row_on))
            def _():
                for j in range(16):
                    maskrow[pl.ds(j * 16, 16)] = zeros_f
                pltpu.sync_copy(maskrow, mask_out.at[pl.ds(mrow_off, _N)])

            return carry

        lax.fori_loop(0, _ROWS_W, rowbody, 0)

        @pl.when(wid == s)
        def _():
            def zbody(z, carry):
                featbuf[pl.ds(z * 16, 16)] = zeros_f
                return carry

            lax.fori_loop(0, 128, zbody, 0)

            p0 = n_s * n_s
            ap = pl.multiple_of((s * _OBS + p0) & (-8), 8)
            offp = (s * _OBS + p0) - ap
            pltpu.sync_copy(obs.at[pl.ds(ap, 896)], posbuf)
            for j in range(16):
                node = j * 16 + iota
                nm = node < n_s
                for f in range(3):
                    vals = plsc.load_gather(posbuf, [offp + 3 * node + f])
                    vals = jnp.where(nm, vals, 0.0)
                    plsc.store_scatter(featbuf, [8 * node + f], vals)

            t0 = 2 * p0 + 4 * n_s
            at0 = pl.multiple_of((s * _OBS + t0) & (-8), 8)
            offt = (s * _OBS + t0) - at0
            pltpu.sync_copy(obs.at[pl.ds(at0, 384)], trbuf)
            tvals = plsc.load_gather(trbuf, [offt + iota])
            plsc.store_scatter(featbuf, [8 * iota + 4], tvals,
                               mask=iota < 5)
            for j in range(16):
                node = j * 16 + iota
                vals = plsc.load_gather(trbuf, [offt + 5 + node])
                vals = jnp.where(node < n_s, vals, 0.0)
                plsc.store_scatter(featbuf, [8 * node + 3], vals)

            nvals = jnp.where(iota == 0, n_f, 0.0)
            plsc.store_scatter(featbuf, [iota * 0 + 5], nvals,
                               mask=iota == 0)
            pltpu.sync_copy(featbuf,
                            feat_out.at[pl.ds(s * _N * 8, _N * 8)])


@functools.cache
def _get_sc_extract():
    return functools.partial(
        pl.kernel,
        mesh=plsc.VectorSubcoreMesh(core_axis_name="c",
                                    subcore_axis_name="s"),
        compiler_params=pltpu.CompilerParams(needs_layout_passes=False),
        out_type=[
            jax.ShapeDtypeStruct((_B * _N * _N,), jnp.float32),
            jax.ShapeDtypeStruct((_B * _N * 8,), jnp.float32),
        ],
        scratch_types=[
            pltpu.VMEM((_CHUNK,), jnp.float32),
            pltpu.VMEM((_B, 16), jnp.int32),
            pltpu.VMEM_SHARED((_NS, _B, 16), jnp.int32),
            pltpu.VMEM((_NS, _B, 16), jnp.int32),
            pltpu.VMEM((16,), jnp.int32),
            pltpu.VMEM((384,), jnp.float32),
            pltpu.VMEM((_N,), jnp.float32),
            pltpu.VMEM((896,), jnp.float32),
            pltpu.VMEM((384,), jnp.float32),
            pltpu.VMEM((_N * 8,), jnp.float32),
        ],
    )(_sc_extract_kernel)


def _tdot(a, b):
    return lax.dot_general(a, b, (((0,), (0,)), ((), ())),
                           preferred_element_type=jnp.float32)


def _dot(a, b):
    return lax.dot_general(a, b, (((1,), (0,)), ((), ())),
                           preferred_element_type=jnp.float32)


def _lnorm(x, g, b):
    m = jnp.mean(x, axis=-1, keepdims=True)
    v = jnp.mean((x - m) * (x - m), axis=-1, keepdims=True)
    return (x - m) / jnp.sqrt(v + 1e-5) * g + b


def _leaky(x):
    return jnp.where(x >= 0.0, x, 0.2 * x)


def _elu(x):
    return jnp.where(x > 0.0, x, jnp.exp(x) - 1.0)


def _attn(mask_f, asrc_col, adst_row, h_feat):
    mb = mask_f > 0.5
    logit = _leaky(asrc_col + adst_row)
    mx = jnp.max(jnp.where(mb, logit, -1e30), axis=0, keepdims=True)
    p = jnp.where(mb, jnp.exp(logit - mx), 0.0)
    ssum = jnp.sum(p, axis=0, keepdims=True)
    return _tdot(p / (ssum + 1e-16), h_feat)


def _tc_gnn_kernel(mask_ref, feat_ref,
                   enc_wt, enc_b, enc_g, enc_beta,
                   proj_wt, proj_b,
                   gat1_wt, as_bd, ad_bd, gat1_b, n1_g, n1_b,
                   gat2_wt, as2, ad2, gat2_b, n2_g, n2_b,
                   tr_wt, tr_b, tr_g, tr_beta,
                   fus_wt, fus_b, fus_g, fus_beta,
                   out_ref):
    mask_f = mask_ref[0]
    feat = feat_ref[0]
    n_f = feat_ref[0, 0, 5]

    x_raw = feat[:, 0:4]
    x = _lnorm(jax.nn.relu(_dot(x_raw, enc_wt[...]) + enc_b[...]),
               enc_g[...], enc_beta[...])
    ident = _dot(x, proj_wt[...]) + proj_b[...]

    ii = lax.broadcasted_iota(jnp.int32, (_N, _N), 0)
    jj = lax.broadcasted_iota(jnp.int32, (_N, _N), 1)
    eye = jnp.where(ii == jj, 1.0, 0.0)

    h1 = _dot(x, gat1_wt[...])
    asrc4 = _dot(h1, as_bd[...])
    adst4t = _tdot(_dot(h1, ad_bd[...]), eye)
    heads = []
    for h in range(4):
        heads.append(_attn(mask_f, asrc4[:, h:h + 1],
                           adst4t[h:h + 1, :],
                           h1[:, 64 * h:64 * h + 64]))
    g1 = jnp.concatenate(heads, axis=1) + gat1_b[...]
    g1 = _elu(_lnorm(g1, n1_g[...], n1_b[...]))

    h2 = _dot(g1, gat2_wt[...])
    a2s = _dot(h2, as2[...])
    a2dt = _tdot(_dot(h2, ad2[...]), eye)
    o2 = _attn(mask_f, a2s, a2dt, h2) + gat2_b[...]
    o2 = _lnorm(o2, n2_g[...], n2_b[...])
    gfin = _elu(o2 + ident)

    node_col = lax.broadcasted_iota(jnp.int32, (_N, 1), 0).astype(jnp.float32)
    nm = jnp.where(node_col < n_f, 1.0, 0.0)
    pooled = _tdot(nm, gfin) / n_f

    traffic = feat[0:5, 4:5]
    tf = _lnorm(jax.nn.relu(_tdot(traffic, tr_wt[...]) + tr_b[...]),
                tr_g[...], tr_beta[...])

    comb = jnp.concatenate([pooled, tf], axis=1)
    out = _lnorm(jax.nn.relu(_dot(comb, fus_wt[...]) + fus_b[...]),
                 fus_g[...], fus_beta[...])
    out_ref[0] = out


def _full_spec(shape):
    nd = len(shape)
    return pl.BlockSpec(shape, lambda s, _n=nd: (0,) * _n)


def kernel(observations, params):
    p = params
    maskflat, featflat = _get_sc_extract()(observations.reshape(-1))
    mask = maskflat.reshape(_B, _N, _N)
    feat = featflat.reshape(_B, _N, 8)

    heads = 4
    as_bd = (jnp.eye(heads, dtype=jnp.float32)[:, None, :]
             * p['gat1_as'][:, :, None]).reshape(heads * 64, heads)
    ad_bd = (jnp.eye(heads, dtype=jnp.float32)[:, None, :]
             * p['gat1_ad'][:, :, None]).reshape(heads * 64, heads)

    weights = [
        p['enc_W'].T,
        p['enc_b'][None, :], p['enc_g'][None, :], p['enc_beta'][None, :],
        p['proj_W'].T,
        p['proj_b'][None, :],
        p['gat1_W'].T,
        as_bd, ad_bd,
        p['gat1_b'][None, :], p['n1_g'][None, :], p['n1_b'][None, :],
        p['gat2_W'].T,
        p['gat2_as'].T, p['gat2_ad'].T,
        p['gat2_b'][None, :], p['n2_g'][None, :], p['n2_b'][None, :],
        p['tr_W'].T,
        p['tr_b'][None, :], p['tr_g'][None, :], p['tr_beta'][None, :],
        p['fus_W'].T,
        p['fus_b'][None, :], p['fus_g'][None, :], p['fus_beta'][None, :],
    ]

    in_specs = [
        pl.BlockSpec((1, _N, _N), lambda s: (s, 0, 0)),
        pl.BlockSpec((1, _N, 8), lambda s: (s, 0, 0)),
    ] + [_full_spec(w.shape) for w in weights]

    out = pl.pallas_call(
        _tc_gnn_kernel,
        grid=(_B,),
        in_specs=in_specs,
        out_specs=pl.BlockSpec((1, 1, _N), lambda s: (s, 0, 0)),
        out_shape=jax.ShapeDtypeStruct((_B, 1, _N), jnp.float32),
    )(mask, feat, *weights)
    return out.reshape(_B, _N)


# speedup vs baseline: 502.2989x; 1.1772x over previous
import functools

import jax
import jax.numpy as jnp
from jax import lax
from jax.experimental import pallas as pl
from jax.experimental.pallas import tpu as pltpu
from jax.experimental.pallas import tpu_sc as plsc

_B = 4
_OBS = 262144
_N = 256
_NC = 2
_NS = 16
_NW = _NC * _NS
_CHUNK = _OBS // _NS
_ROWS_W = _N // _NW
_INLEN = 2432


def _sc_extract_kernel(obs, mask_out, feat_out,
                       cbuf0, cbuf1, cntmat, shared, allcnt,
                       inb0, inb1, mb0, mb1, posbuf, trbuf, featbuf,
                       csem, isem, osem0, osem1):
    cid = lax.axis_index("c")
    sid = lax.axis_index("s")
    wid = sid * _NC + cid
    iota = lax.iota(jnp.int32, 16)
    zeros_i = jnp.zeros((16,), jnp.int32)
    ones_i = jnp.ones((16,), jnp.int32)
    zeros_f = jnp.zeros((16,), jnp.float32)

    cbufs = [cbuf0, cbuf1]
    for s in range(_B):
        buf = cbufs[s % 2]
        pltpu.sync_copy(obs.at[pl.ds(s * _OBS + sid * _CHUNK, _CHUNK)],
                        buf)

        def cbody(i, acc, _buf=buf):
            for u in range(8):
                v = _buf[pl.ds(i * 128 + u * 16, 16)]
                acc = acc + jnp.where(v != 0.0, ones_i, zeros_i)
            return acc

        cntmat[pl.ds(s * 16, 16)] = lax.fori_loop(0, _CHUNK // 128,
                                                  cbody, zeros_i)

    pltpu.sync_copy(cntmat, shared.at[pl.ds(sid * _B * 16, _B * 16)])
    plsc.subcore_barrier()
    pltpu.sync_copy(shared, allcnt)

    n_list = []
    for s in range(_B):
        tot = zeros_i
        for r in range(_NS):
            tot = tot + allcnt[pl.ds((r * _B + s) * 16, 16)]
        nz = tot[0]
        for k in range(1, 16):
            nz = nz + tot[k]
        r0 = jnp.int32(0)
        for b in [512, 256, 128, 64, 32, 16, 8, 4, 2, 1]:
            t = r0 + b
            r0 = jnp.where(t * t <= nz, t, r0)
        n_list.append(jnp.minimum(jnp.maximum(r0 >> 1, 5), _N))

    base = wid * _ROWS_W
    inbufs = [inb0, inb1]
    mbufs = [mb0, mb1]

    def _in_abase(s):
        return pl.multiple_of((s * _OBS + base * n_list[s]) & (-8), 8)

    osems = [osem0, osem1]
    pend_in = pltpu.async_copy(obs.at[pl.ds(_in_abase(0), _INLEN)],
                               inbufs[0], isem)
    out_copies = [None] * _B
    for s in range(_B):
        n_s = n_list[s]
        off0 = (s * _OBS + base * n_s) - _in_abase(s)
        pend_in.wait()
        if s + 1 < _B:
            pend_in = pltpu.async_copy(
                obs.at[pl.ds(_in_abase(s + 1), _INLEN)],
                inbufs[(s + 1) % 2], isem)
        if s >= 2:
            out_copies[s - 2].wait()
        ib = inbufs[s % 2]
        mb = mbufs[s % 2]
        for k in range(_ROWS_W):
            rowvec = iota * 0 + (base + k)
            roff = off0 + k * n_s
            for j in range(16):
                col = j * 16 + iota
                vals = plsc.load_gather(ib, [roff + col])
                act = (vals != 0.0) & (col < n_s) & (rowvec < n_s)
                mb[pl.ds(k * _N + j * 16, 16)] = jnp.where(act, 1.0, 0.0)
        mout = pl.multiple_of(s * _N * _N + base * _N, 8)
        out_copies[s] = pltpu.async_copy(
            mb, mask_out.at[pl.ds(mout, _ROWS_W * _N)],
            osems[s % 2])
    out_copies[_B - 2].wait()
    out_copies[_B - 1].wait()

    for s in range(_B):
        n_s = n_list[s]
        n_f = n_s.astype(jnp.float32)

        @pl.when(wid == s)
        def _():
            def zbody(z, carry):
                featbuf[pl.ds(z * 16, 16)] = zeros_f
                return carry

            lax.fori_loop(0, 128, zbody, 0)

            p0 = n_s * n_s
            ap = pl.multiple_of((s * _OBS + p0) & (-8), 8)
            offp = (s * _OBS + p0) - ap
            pltpu.sync_copy(obs.at[pl.ds(ap, 896)], posbuf)
            for j in range(16):
                node = j * 16 + iota
                nm = node < n_s
                for f in range(3):
                    vals = plsc.load_gather(posbuf, [offp + 3 * node + f])
                    vals = jnp.where(nm, vals, 0.0)
                    plsc.store_scatter(featbuf, [8 * node + f], vals)

            t0 = 2 * p0 + 4 * n_s
            at0 = pl.multiple_of((s * _OBS + t0) & (-8), 8)
            offt = (s * _OBS + t0) - at0
            pltpu.sync_copy(obs.at[pl.ds(at0, 384)], trbuf)
            tvals = plsc.load_gather(trbuf, [offt + iota])
            plsc.store_scatter(featbuf, [8 * iota + 4], tvals,
                               mask=iota < 5)
            for j in range(16):
                node = j * 16 + iota
                vals = plsc.load_gather(trbuf, [offt + 5 + node])
                vals = jnp.where(node < n_s, vals, 0.0)
                plsc.store_scatter(featbuf, [8 * node + 3], vals)

            nvals = jnp.where(iota == 0, n_f, 0.0)
            plsc.store_scatter(featbuf, [iota * 0 + 5], nvals,
                               mask=iota == 0)
            pltpu.sync_copy(featbuf,
                            feat_out.at[pl.ds(s * _N * 8, _N * 8)])


@functools.cache
def _get_sc_extract():
    return functools.partial(
        pl.kernel,
        mesh=plsc.VectorSubcoreMesh(core_axis_name="c",
                                    subcore_axis_name="s"),
        compiler_params=pltpu.CompilerParams(needs_layout_passes=False),
        out_type=[
            jax.ShapeDtypeStruct((_B * _N * _N,), jnp.float32),
            jax.ShapeDtypeStruct((_B * _N * 8,), jnp.float32),
        ],
        scratch_types=[
            pltpu.VMEM((_CHUNK,), jnp.float32),
            pltpu.VMEM((_CHUNK,), jnp.float32),
            pltpu.VMEM((_B * 16,), jnp.int32),
            pltpu.VMEM_SHARED((_NS * _B * 16,), jnp.int32),
            pltpu.VMEM((_NS * _B * 16,), jnp.int32),
            pltpu.VMEM((_INLEN,), jnp.float32),
            pltpu.VMEM((_INLEN,), jnp.float32),
            pltpu.VMEM((_ROWS_W * _N,), jnp.float32),
            pltpu.VMEM((_ROWS_W * _N,), jnp.float32),
            pltpu.VMEM((896,), jnp.float32),
            pltpu.VMEM((384,), jnp.float32),
            pltpu.VMEM((_N * 8,), jnp.float32),
            pltpu.SemaphoreType.DMA,
            pltpu.SemaphoreType.DMA,
            pltpu.SemaphoreType.DMA,
            pltpu.SemaphoreType.DMA,
        ],
    )(_sc_extract_kernel)


def _tdot(a, b):
    return lax.dot_general(a, b, (((0,), (0,)), ((), ())),
                           preferred_element_type=jnp.float32)


def _dot(a, b):
    return lax.dot_general(a, b, (((1,), (0,)), ((), ())),
                           preferred_element_type=jnp.float32)


def _lnorm(x, g, b):
    m = jnp.mean(x, axis=-1, keepdims=True)
    v = jnp.mean((x - m) * (x - m), axis=-1, keepdims=True)
    return (x - m) / jnp.sqrt(v + 1e-5) * g + b


def _leaky(x):
    return jnp.where(x >= 0.0, x, 0.2 * x)


def _elu(x):
    return jnp.where(x > 0.0, x, jnp.exp(x) - 1.0)


def _attn(mask_f, asrc_col, adst_row, h_feat):
    mb = mask_f > 0.5
    logit = _leaky(asrc_col + adst_row)
    mx = jnp.max(jnp.where(mb, logit, -1e30), axis=0, keepdims=True)
    p = jnp.where(mb, jnp.exp(logit - mx), 0.0)
    ssum = jnp.sum(p, axis=0, keepdims=True)
    return _tdot(p / (ssum + 1e-16), h_feat)


def _tc_gnn_kernel(mask_ref, feat_ref,
                   enc_wt, enc_b, enc_g, enc_beta,
                   proj_wt, proj_b,
                   gat1_wt, as_bd, ad_bd, gat1_b, n1_g, n1_b,
                   gat2_wt, as2, ad2, gat2_b, n2_g, n2_b,
                   tr_wt, tr_b, tr_g, tr_beta,
                   fus_wt, fus_b, fus_g, fus_beta,
                   out_ref):
    mask_f = mask_ref[0]
    feat = feat_ref[0]
    n_f = feat_ref[0, 0, 5]

    x_raw = feat[:, 0:4]
    x = _lnorm(jax.nn.relu(_dot(x_raw, enc_wt[...]) + enc_b[...]),
               enc_g[...], enc_beta[...])
    ident = _dot(x, proj_wt[...]) + proj_b[...]

    ii = lax.broadcasted_iota(jnp.int32, (_N, _N), 0)
    jj = lax.broadcasted_iota(jnp.int32, (_N, _N), 1)
    eye = jnp.where(ii == jj, 1.0, 0.0)

    h1 = _dot(x, gat1_wt[...])
    asrc4 = _dot(h1, as_bd[...])
    adst4t = _tdot(_dot(h1, ad_bd[...]), eye)
    heads = []
    for h in range(4):
        heads.append(_attn(mask_f, asrc4[:, h:h + 1],
                           adst4t[h:h + 1, :],
                           h1[:, 64 * h:64 * h + 64]))
    g1 = jnp.concatenate(heads, axis=1) + gat1_b[...]
    g1 = _elu(_lnorm(g1, n1_g[...], n1_b[...]))

    h2 = _dot(g1, gat2_wt[...])
    a2s = _dot(h2, as2[...])
    a2dt = _tdot(_dot(h2, ad2[...]), eye)
    o2 = _attn(mask_f, a2s, a2dt, h2) + gat2_b[...]
    o2 = _lnorm(o2, n2_g[...], n2_b[...])
    gfin = _elu(o2 + ident)

    node_col = lax.broadcasted_iota(jnp.int32, (_N, 1), 0).astype(jnp.float32)
    nm = jnp.where(node_col < n_f, 1.0, 0.0)
    pooled = _tdot(nm, gfin) / n_f

    traffic = feat[0:5, 4:5]
    tf = _lnorm(jax.nn.relu(_tdot(traffic, tr_wt[...]) + tr_b[...]),
                tr_g[...], tr_beta[...])

    comb = jnp.concatenate([pooled, tf], axis=1)
    out = _lnorm(jax.nn.relu(_dot(comb, fus_wt[...]) + fus_b[...]),
                 fus_g[...], fus_beta[...])
    out_ref[0] = out


def _full_spec(shape):
    nd = len(shape)
    return pl.BlockSpec(shape, lambda s, _n=nd: (0,) * _n)


def kernel(observations, params):
    p = params
    maskflat, featflat = _get_sc_extract()(observations.reshape(-1))
    mask = maskflat.reshape(_B, _N, _N)
    feat = featflat.reshape(_B, _N, 8)

    heads = 4
    as_bd = (jnp.eye(heads, dtype=jnp.float32)[:, None, :]
             * p['gat1_as'][:, :, None]).reshape(heads * 64, heads)
    ad_bd = (jnp.eye(heads, dtype=jnp.float32)[:, None, :]
             * p['gat1_ad'][:, :, None]).reshape(heads * 64, heads)

    weights = [
        p['enc_W'].T,
        p['enc_b'][None, :], p['enc_g'][None, :], p['enc_beta'][None, :],
        p['proj_W'].T,
        p['proj_b'][None, :],
        p['gat1_W'].T,
        as_bd, ad_bd,
        p['gat1_b'][None, :], p['n1_g'][None, :], p['n1_b'][None, :],
        p['gat2_W'].T,
        p['gat2_as'].T, p['gat2_ad'].T,
        p['gat2_b'][None, :], p['n2_g'][None, :], p['n2_b'][None, :],
        p['tr_W'].T,
        p['tr_b'][None, :], p['tr_g'][None, :], p['tr_beta'][None, :],
        p['fus_W'].T,
        p['fus_b'][None, :], p['fus_g'][None, :], p['fus_beta'][None, :],
    ]

    in_specs = [
        pl.BlockSpec((1, _N, _N), lambda s: (s, 0, 0)),
        pl.BlockSpec((1, _N, 8), lambda s: (s, 0, 0)),
    ] + [_full_spec(w.shape) for w in weights]

    out = pl.pallas_call(
        _tc_gnn_kernel,
        grid=(_B,),
        in_specs=in_specs,
        out_specs=pl.BlockSpec((1, 1, _N), lambda s: (s, 0, 0)),
        out_shape=jax.ShapeDtypeStruct((_B, 1, _N), jnp.float32),
    )(mask, feat, *weights)
    return out.reshape(_B, _N)


# speedup vs baseline: 528.3252x; 1.0518x over previous
import functools

import jax
import jax.numpy as jnp
from jax import lax
from jax.experimental import pallas as pl
from jax.experimental.pallas import tpu as pltpu
from jax.experimental.pallas import tpu_sc as plsc

_B = 4
_OBS = 262144
_N = 256
_NC = 2
_NS = 16
_NW = _NC * _NS
_CHUNK = _OBS // _NS
_ROWS_W = _N // _NW
_INLEN = 2432


def _sc_extract_kernel(obs, mask_out, feat_out,
                       cbuf0, cbuf1, cntmat, shared, allcnt,
                       inb0, inb1, mb0, mb1, posbuf, trbuf, featbuf,
                       csem, isem, osem0, osem1):
    cid = lax.axis_index("c")
    sid = lax.axis_index("s")
    wid = sid * _NC + cid
    iota = lax.iota(jnp.int32, 16)
    zeros_i = jnp.zeros((16,), jnp.int32)
    ones_i = jnp.ones((16,), jnp.int32)
    zeros_f = jnp.zeros((16,), jnp.float32)

    cbufs = [cbuf0, cbuf1]
    pend = pltpu.async_copy(obs.at[pl.ds(sid * _CHUNK, _CHUNK)],
                            cbufs[0], csem)
    for s in range(_B):
        buf = cbufs[s % 2]
        pend.wait()
        if s + 1 < _B:
            pend = pltpu.async_copy(
                obs.at[pl.ds((s + 1) * _OBS + sid * _CHUNK, _CHUNK)],
                cbufs[(s + 1) % 2], csem)

        def cbody(i, acc, _buf=buf):
            for u in range(8):
                v = _buf[pl.ds(i * 128 + u * 16, 16)]
                acc = acc + jnp.where(v != 0.0, ones_i, zeros_i)
            return acc

        cntmat[pl.ds(s * 16, 16)] = lax.fori_loop(0, _CHUNK // 128,
                                                  cbody, zeros_i)

    pltpu.sync_copy(cntmat, shared.at[pl.ds(sid * _B * 16, _B * 16)])
    plsc.subcore_barrier()
    pltpu.sync_copy(shared, allcnt)

    n_list = []
    for s in range(_B):
        tot = zeros_i
        for r in range(_NS):
            tot = tot + allcnt[pl.ds((r * _B + s) * 16, 16)]
        nz = tot[0]
        for k in range(1, 16):
            nz = nz + tot[k]
        r0 = jnp.int32(0)
        for b in [512, 256, 128, 64, 32, 16, 8, 4, 2, 1]:
            t = r0 + b
            r0 = jnp.where(t * t <= nz, t, r0)
        n_list.append(jnp.minimum(jnp.maximum(r0 >> 1, 5), _N))

    base = wid * _ROWS_W
    inbufs = [inb0, inb1]
    mbufs = [mb0, mb1]

    def _in_abase(s):
        return pl.multiple_of((s * _OBS + base * n_list[s]) & (-8), 8)

    osems = [osem0, osem1]
    pend_in = pltpu.async_copy(obs.at[pl.ds(_in_abase(0), _INLEN)],
                               inbufs[0], isem)
    out_copies = [None] * _B
    for s in range(_B):
        n_s = n_list[s]
        off0 = (s * _OBS + base * n_s) - _in_abase(s)
        pend_in.wait()
        if s + 1 < _B:
            pend_in = pltpu.async_copy(
                obs.at[pl.ds(_in_abase(s + 1), _INLEN)],
                inbufs[(s + 1) % 2], isem)
        if s >= 2:
            out_copies[s - 2].wait()
        ib = inbufs[s % 2]
        mb = mbufs[s % 2]
        for k in range(_ROWS_W):
            rowvec = iota * 0 + (base + k)
            roff = off0 + k * n_s
            for j in range(16):
                col = j * 16 + iota
                vals = plsc.load_gather(ib, [roff + col])
                act = (vals != 0.0) & (col < n_s) & (rowvec < n_s)
                mb[pl.ds(k * _N + j * 16, 16)] = jnp.where(act, 1.0, 0.0)
        mout = pl.multiple_of(s * _N * _N + base * _N, 8)
        out_copies[s] = pltpu.async_copy(
            mb, mask_out.at[pl.ds(mout, _ROWS_W * _N)],
            osems[s % 2])
    out_copies[_B - 2].wait()
    out_copies[_B - 1].wait()

    for s in range(_B):
        n_s = n_list[s]
        n_f = n_s.astype(jnp.float32)

        @pl.when(wid == s)
        def _():
            def zbody(z, carry):
                featbuf[pl.ds(z * 16, 16)] = zeros_f
                return carry

            lax.fori_loop(0, 128, zbody, 0)

            p0 = n_s * n_s
            ap = pl.multiple_of((s * _OBS + p0) & (-8), 8)
            offp = (s * _OBS + p0) - ap
            pltpu.sync_copy(obs.at[pl.ds(ap, 896)], posbuf)
            for j in range(16):
                node = j * 16 + iota
                nm = node < n_s
                for f in range(3):
                    vals = plsc.load_gather(posbuf, [offp + 3 * node + f])
                    vals = jnp.where(nm, vals, 0.0)
                    plsc.store_scatter(featbuf, [8 * node + f], vals)

            t0 = 2 * p0 + 4 * n_s
            at0 = pl.multiple_of((s * _OBS + t0) & (-8), 8)
            offt = (s * _OBS + t0) - at0
            pltpu.sync_copy(obs.at[pl.ds(at0, 384)], trbuf)
            tvals = plsc.load_gather(trbuf, [offt + iota])
            plsc.store_scatter(featbuf, [8 * iota + 4], tvals,
                               mask=iota < 5)
            for j in range(16):
                node = j * 16 + iota
                vals = plsc.load_gather(trbuf, [offt + 5 + node])
                vals = jnp.where(node < n_s, vals, 0.0)
                plsc.store_scatter(featbuf, [8 * node + 3], vals)

            nvals = jnp.where(iota == 0, n_f, 0.0)
            plsc.store_scatter(featbuf, [iota * 0 + 5], nvals,
                               mask=iota == 0)
            pltpu.sync_copy(featbuf,
                            feat_out.at[pl.ds(s * _N * 8, _N * 8)])


@functools.cache
def _get_sc_extract():
    return functools.partial(
        pl.kernel,
        mesh=plsc.VectorSubcoreMesh(core_axis_name="c",
                                    subcore_axis_name="s"),
        compiler_params=pltpu.CompilerParams(needs_layout_passes=False),
        out_type=[
            jax.ShapeDtypeStruct((_B * _N * _N,), jnp.float32),
            jax.ShapeDtypeStruct((_B * _N * 8,), jnp.float32),
        ],
        scratch_types=[
            pltpu.VMEM((_CHUNK,), jnp.float32),
            pltpu.VMEM((_CHUNK,), jnp.float32),
            pltpu.VMEM((_B * 16,), jnp.int32),
            pltpu.VMEM_SHARED((_NS * _B * 16,), jnp.int32),
            pltpu.VMEM((_NS * _B * 16,), jnp.int32),
            pltpu.VMEM((_INLEN,), jnp.float32),
            pltpu.VMEM((_INLEN,), jnp.float32),
            pltpu.VMEM((_ROWS_W * _N,), jnp.float32),
            pltpu.VMEM((_ROWS_W * _N,), jnp.float32),
            pltpu.VMEM((896,), jnp.float32),
            pltpu.VMEM((384,), jnp.float32),
            pltpu.VMEM((_N * 8,), jnp.float32),
            pltpu.SemaphoreType.DMA,
            pltpu.SemaphoreType.DMA,
            pltpu.SemaphoreType.DMA,
            pltpu.SemaphoreType.DMA,
        ],
    )(_sc_extract_kernel)


def _tdot(a, b):
    return lax.dot_general(a, b, (((0,), (0,)), ((), ())),
                           preferred_element_type=jnp.float32)


def _dot(a, b):
    return lax.dot_general(a, b, (((1,), (0,)), ((), ())),
                           preferred_element_type=jnp.float32)


def _lnorm(x, g, b):
    m = jnp.mean(x, axis=-1, keepdims=True)
    v = jnp.mean((x - m) * (x - m), axis=-1, keepdims=True)
    return (x - m) * lax.rsqrt(v + 1e-5) * g + b


def _leaky(x):
    return jnp.where(x >= 0.0, x, 0.2 * x)


def _elu(x):
    return jnp.where(x > 0.0, x, jnp.exp(x) - 1.0)


def _attn(negmask, asrc_col, adst_row, h_feat):
    logit = _leaky(asrc_col + adst_row) + negmask
    mx = jnp.max(logit, axis=0, keepdims=True)
    p = jnp.exp(logit - mx)
    rsum = 1.0 / (jnp.sum(p, axis=0, keepdims=True) + 1e-16)
    return _tdot(p * rsum, h_feat)


def _tc_gnn_kernel(mask_ref, feat_ref,
                   enc_wt, enc_b, enc_g, enc_beta,
                   proj_wt, proj_b,
                   gat1_wt, as_bd, ad_bd, gat1_b, n1_g, n1_b,
                   gat2_wt, as2, ad2, gat2_b, n2_g, n2_b,
                   tr_wt, tr_b, tr_g, tr_beta,
                   fus_wt, fus_b, fus_g, fus_beta,
                   out_ref):
    mask_f = mask_ref[0]
    negmask = jnp.where(mask_f > 0.5, 0.0, -1e30)
    feat = feat_ref[0]
    n_f = feat_ref[0, 0, 5]

    x_raw = feat[:, 0:4]
    x = _lnorm(jax.nn.relu(_dot(x_raw, enc_wt[...]) + enc_b[...]),
               enc_g[...], enc_beta[...])
    ident = _dot(x, proj_wt[...]) + proj_b[...]

    ii = lax.broadcasted_iota(jnp.int32, (_N, _N), 0)
    jj = lax.broadcasted_iota(jnp.int32, (_N, _N), 1)
    eye = jnp.where(ii == jj, 1.0, 0.0)

    h1 = _dot(x, gat1_wt[...])
    asrc4 = _dot(h1, as_bd[...])
    adst4t = _tdot(_dot(h1, ad_bd[...]), eye)
    heads = []
    for h in range(4):
        heads.append(_attn(negmask, asrc4[:, h:h + 1],
                           adst4t[h:h + 1, :],
                           h1[:, 64 * h:64 * h + 64]))
    g1 = jnp.concatenate(heads, axis=1) + gat1_b[...]
    g1 = _elu(_lnorm(g1, n1_g[...], n1_b[...]))

    h2 = _dot(g1, gat2_wt[...])
    a2s = _dot(h2, as2[...])
    a2dt = _tdot(_dot(h2, ad2[...]), eye)
    o2 = _attn(negmask, a2s, a2dt, h2) + gat2_b[...]
    o2 = _lnorm(o2, n2_g[...], n2_b[...])
    gfin = _elu(o2 + ident)

    node_col = lax.broadcasted_iota(jnp.int32, (_N, 1), 0).astype(jnp.float32)
    nm = jnp.where(node_col < n_f, 1.0, 0.0)
    pooled = _tdot(nm, gfin) / n_f

    traffic = feat[0:5, 4:5]
    tf = _lnorm(jax.nn.relu(_tdot(traffic, tr_wt[...]) + tr_b[...]),
                tr_g[...], tr_beta[...])

    comb = jnp.concatenate([pooled, tf], axis=1)
    out = _lnorm(jax.nn.relu(_dot(comb, fus_wt[...]) + fus_b[...]),
                 fus_g[...], fus_beta[...])
    out_ref[0] = out


def _full_spec(shape):
    nd = len(shape)
    return pl.BlockSpec(shape, lambda s, _n=nd: (0,) * _n)


def kernel(observations, params):
    p = params
    maskflat, featflat = _get_sc_extract()(observations.reshape(-1))
    mask = maskflat.reshape(_B, _N, _N)
    feat = featflat.reshape(_B, _N, 8)

    heads = 4
    as_bd = (jnp.eye(heads, dtype=jnp.float32)[:, None, :]
             * p['gat1_as'][:, :, None]).reshape(heads * 64, heads)
    ad_bd = (jnp.eye(heads, dtype=jnp.float32)[:, None, :]
             * p['gat1_ad'][:, :, None]).reshape(heads * 64, heads)

    weights = [
        p['enc_W'].T,
        p['enc_b'][None, :], p['enc_g'][None, :], p['enc_beta'][None, :],
        p['proj_W'].T,
        p['proj_b'][None, :],
        p['gat1_W'].T,
        as_bd, ad_bd,
        p['gat1_b'][None, :], p['n1_g'][None, :], p['n1_b'][None, :],
        p['gat2_W'].T,
        p['gat2_as'].T, p['gat2_ad'].T,
        p['gat2_b'][None, :], p['n2_g'][None, :], p['n2_b'][None, :],
        p['tr_W'].T,
        p['tr_b'][None, :], p['tr_g'][None, :], p['tr_beta'][None, :],
        p['fus_W'].T,
        p['fus_b'][None, :], p['fus_g'][None, :], p['fus_beta'][None, :],
    ]

    in_specs = [
        pl.BlockSpec((1, _N, _N), lambda s: (s, 0, 0)),
        pl.BlockSpec((1, _N, 8), lambda s: (s, 0, 0)),
    ] + [_full_spec(w.shape) for w in weights]

    out = pl.pallas_call(
        _tc_gnn_kernel,
        grid=(_B,),
        in_specs=in_specs,
        out_specs=pl.BlockSpec((1, 1, _N), lambda s: (s, 0, 0)),
        out_shape=jax.ShapeDtypeStruct((_B, 1, _N), jnp.float32),
    )(mask, feat, *weights)
    return out.reshape(_B, _N)


# speedup vs baseline: 551.5433x; 1.0439x over previous
import functools

import jax
import jax.numpy as jnp
from jax import lax
from jax.experimental import pallas as pl
from jax.experimental.pallas import tpu as pltpu
from jax.experimental.pallas import tpu_sc as plsc

_B = 4
_OBS = 262144
_N = 256
_NC = 2
_NS = 16
_NW = _NC * _NS
_CHUNK = _OBS // _NS
_ROWS_W = _N // _NW
_INLEN = 2432


def _sc_extract_kernel(obs, mask_out, feat_out,
                       cbuf0, cbuf1, cntmat, shared, allcnt,
                       inb0, inb1, mb0, mb1, posbuf, trbuf, featbuf,
                       csem, isem, osem0, osem1):
    cid = lax.axis_index("c")
    sid = lax.axis_index("s")
    wid = sid * _NC + cid
    iota = lax.iota(jnp.int32, 16)
    zeros_i = jnp.zeros((16,), jnp.int32)
    ones_i = jnp.ones((16,), jnp.int32)
    zeros_f = jnp.zeros((16,), jnp.float32)

    cbufs = [cbuf0, cbuf1]
    pend = pltpu.async_copy(obs.at[pl.ds(sid * _CHUNK, _CHUNK)],
                            cbufs[0], csem)
    for s in range(_B):
        buf = cbufs[s % 2]
        pend.wait()
        if s + 1 < _B:
            pend = pltpu.async_copy(
                obs.at[pl.ds((s + 1) * _OBS + sid * _CHUNK, _CHUNK)],
                cbufs[(s + 1) % 2], csem)

        def cbody(i, acc, _buf=buf):
            for u in range(8):
                v = _buf[pl.ds(i * 128 + u * 16, 16)]
                acc = acc + jnp.where(v != 0.0, ones_i, zeros_i)
            return acc

        cntmat[pl.ds(s * 16, 16)] = lax.fori_loop(0, _CHUNK // 128,
                                                  cbody, zeros_i)

    pltpu.sync_copy(cntmat, shared.at[pl.ds(sid * _B * 16, _B * 16)])
    plsc.subcore_barrier()
    pltpu.sync_copy(shared, allcnt)

    n_list = []
    for s in range(_B):
        tot = zeros_i
        for r in range(_NS):
            tot = tot + allcnt[pl.ds((r * _B + s) * 16, 16)]
        nz = tot[0]
        for k in range(1, 16):
            nz = nz + tot[k]
        r0 = jnp.int32(0)
        for b in [512, 256, 128, 64, 32, 16, 8, 4, 2, 1]:
            t = r0 + b
            r0 = jnp.where(t * t <= nz, t, r0)
        n_list.append(jnp.minimum(jnp.maximum(r0 >> 1, 5), _N))

    base = wid * _ROWS_W
    inbufs = [inb0, inb1]
    mbufs = [mb0, mb1]

    def _in_abase(s):
        return pl.multiple_of((s * _OBS + base * n_list[s]) & (-8), 8)

    osems = [osem0, osem1]
    pend_in = pltpu.async_copy(obs.at[pl.ds(_in_abase(0), _INLEN)],
                               inbufs[0], isem)
    out_copies = [None] * _B
    for s in range(_B):
        n_s = n_list[s]
        off0 = (s * _OBS + base * n_s) - _in_abase(s)
        pend_in.wait()
        if s + 1 < _B:
            pend_in = pltpu.async_copy(
                obs.at[pl.ds(_in_abase(s + 1), _INLEN)],
                inbufs[(s + 1) % 2], isem)
        if s >= 2:
            for c in out_copies[s - 2]:
                c.wait()
        ib = inbufs[s % 2]
        mb = mbufs[s % 2]
        for k in range(_ROWS_W):
            rowvec = iota * 0 + (base + k)
            roff = off0 + k * n_s
            for j in range(16):
                col = j * 16 + iota
                vals = plsc.load_gather(ib, [roff + col])
                act = (vals != 0.0) & (col < n_s) & (rowvec < n_s)
                mb[pl.ds((j // 8) * (_ROWS_W * 128) + k * 128
                         + (j % 8) * 16, 16)] = jnp.where(act, 1.0, 0.0)
        half = _ROWS_W * 128
        out_copies[s] = []
        for h in range(2):
            mout = pl.multiple_of(
                s * _N * _N + h * (_N * 128) + base * 128, 8)
            out_copies[s].append(pltpu.async_copy(
                mb.at[pl.ds(h * half, half)],
                mask_out.at[pl.ds(mout, half)], osems[s % 2]))
    for c in out_copies[_B - 2] + out_copies[_B - 1]:
        c.wait()

    for s in range(_B):
        n_s = n_list[s]
        n_f = n_s.astype(jnp.float32)

        @pl.when(wid == s)
        def _():
            def zbody(z, carry):
                featbuf[pl.ds(z * 16, 16)] = zeros_f
                return carry

            lax.fori_loop(64, 128, zbody, 0)

            p0 = n_s * n_s
            ap = pl.multiple_of((s * _OBS + p0) & (-8), 8)
            offp = (s * _OBS + p0) - ap
            pltpu.sync_copy(obs.at[pl.ds(ap, 896)], posbuf)
            for j in range(16):
                node = j * 16 + iota
                nm = node < n_s
                for f in range(3):
                    vals = plsc.load_gather(posbuf, [offp + 3 * node + f])
                    vals = jnp.where(nm, vals, 0.0)
                    featbuf[pl.ds(_N * f + j * 16, 16)] = vals

            t0 = 2 * p0 + 4 * n_s
            at0 = pl.multiple_of((s * _OBS + t0) & (-8), 8)
            offt = (s * _OBS + t0) - at0
            pltpu.sync_copy(obs.at[pl.ds(at0, 384)], trbuf)
            tvals = plsc.load_gather(trbuf, [offt + iota])
            featbuf[pl.ds(_N * 4, 16)] = jnp.where(iota < 5, tvals, 0.0)
            for j in range(16):
                node = j * 16 + iota
                vals = plsc.load_gather(trbuf, [offt + 5 + node])
                vals = jnp.where(node < n_s, vals, 0.0)
                featbuf[pl.ds(_N * 3 + j * 16, 16)] = vals

            featbuf[pl.ds(_N * 5, 16)] = jnp.where(iota == 0, n_f, 0.0)
            pltpu.sync_copy(featbuf,
                            feat_out.at[pl.ds(s * _N * 8, _N * 8)])


@functools.cache
def _get_sc_extract():
    return functools.partial(
        pl.kernel,
        mesh=plsc.VectorSubcoreMesh(core_axis_name="c",
                                    subcore_axis_name="s"),
        compiler_params=pltpu.CompilerParams(needs_layout_passes=False),
        out_type=[
            jax.ShapeDtypeStruct((_B * _N * _N,), jnp.float32),
            jax.ShapeDtypeStruct((_B * _N * 8,), jnp.float32),
        ],
        scratch_types=[
            pltpu.VMEM((_CHUNK,), jnp.float32),
            pltpu.VMEM((_CHUNK,), jnp.float32),
            pltpu.VMEM((_B * 16,), jnp.int32),
            pltpu.VMEM_SHARED((_NS * _B * 16,), jnp.int32),
            pltpu.VMEM((_NS * _B * 16,), jnp.int32),
            pltpu.VMEM((_INLEN,), jnp.float32),
            pltpu.VMEM((_INLEN,), jnp.float32),
            pltpu.VMEM((_ROWS_W * _N,), jnp.float32),
            pltpu.VMEM((_ROWS_W * _N,), jnp.float32),
            pltpu.VMEM((896,), jnp.float32),
            pltpu.VMEM((384,), jnp.float32),
            pltpu.VMEM((_N * 8,), jnp.float32),
            pltpu.SemaphoreType.DMA,
            pltpu.SemaphoreType.DMA,
            pltpu.SemaphoreType.DMA,
            pltpu.SemaphoreType.DMA,
        ],
    )(_sc_extract_kernel)


def _tdot(a, b):
    return lax.dot_general(a, b, (((0,), (0,)), ((), ())),
                           preferred_element_type=jnp.float32)


def _dot(a, b):
    return lax.dot_general(a, b, (((1,), (0,)), ((), ())),
                           preferred_element_type=jnp.float32)


def _lnorm(x, g, b):
    m = jnp.mean(x, axis=-1, keepdims=True)
    v = jnp.mean((x - m) * (x - m), axis=-1, keepdims=True)
    return (x - m) * lax.rsqrt(v + 1e-5) * g + b


def _leaky(x):
    return jnp.where(x >= 0.0, x, 0.2 * x)


def _elu(x):
    return jnp.where(x > 0.0, x, jnp.exp(x) - 1.0)


def _attn(negmask, asrc_col, adst_row, h_feat):
    logit = _leaky(asrc_col + adst_row) + negmask
    mx = jnp.max(logit, axis=0, keepdims=True)
    p = jnp.exp(logit - mx)
    rsum = 1.0 / (jnp.sum(p, axis=0, keepdims=True) + 1e-16)
    return _tdot(p * rsum, h_feat)


def _tc_gnn_kernel(mask_ref, feat_ref,
                   enc_wt, enc_b, enc_g, enc_beta,
                   proj_wt, proj_b,
                   gat1_wt, as_bd, ad_bd, gat1_b, n1_g, n1_b,
                   gat2_wt, as2, ad2, gat2_b, n2_g, n2_b,
                   tr_wt, tr_b, tr_g, tr_beta,
                   fus_wt, fus_b, fus_g, fus_beta,
                   out_ref):
    mm = mask_ref[0]
    mask_f = jnp.concatenate([mm[0], mm[1]], axis=1)
    negmask = jnp.where(mask_f > 0.5, 0.0, -1e30)
    featT = feat_ref[0]
    n_f = feat_ref[0, 5, 0]

    i4 = lax.broadcasted_iota(jnp.int32, (4, 4), 0)
    j4 = lax.broadcasted_iota(jnp.int32, (4, 4), 1)
    eye4 = jnp.where(i4 == j4, 1.0, 0.0)
    x_raw = _tdot(featT[0:4, :], eye4)
    x = _lnorm(jax.nn.relu(_dot(x_raw, enc_wt[...]) + enc_b[...]),
               enc_g[...], enc_beta[...])
    ident = _dot(x, proj_wt[...]) + proj_b[...]

    ii = lax.broadcasted_iota(jnp.int32, (_N, _N), 0)
    jj = lax.broadcasted_iota(jnp.int32, (_N, _N), 1)
    eye = jnp.where(ii == jj, 1.0, 0.0)

    h1 = _dot(x, gat1_wt[...])
    asrc4 = _dot(h1, as_bd[...])
    adst4t = _tdot(_dot(h1, ad_bd[...]), eye)
    heads = []
    for h in range(4):
        heads.append(_attn(negmask, asrc4[:, h:h + 1],
                           adst4t[h:h + 1, :],
                           h1[:, 64 * h:64 * h + 64]))
    g1 = jnp.concatenate(heads, axis=1) + gat1_b[...]
    g1 = _elu(_lnorm(g1, n1_g[...], n1_b[...]))

    h2 = _dot(g1, gat2_wt[...])
    a2s = _dot(h2, as2[...])
    a2dt = _tdot(_dot(h2, ad2[...]), eye)
    o2 = _attn(negmask, a2s, a2dt, h2) + gat2_b[...]
    o2 = _lnorm(o2, n2_g[...], n2_b[...])
    gfin = _elu(o2 + ident)

    node_col = lax.broadcasted_iota(jnp.int32, (_N, 1), 0).astype(jnp.float32)
    nm = jnp.where(node_col < n_f, 1.0, 0.0)
    pooled = _tdot(nm, gfin) / n_f

    traffic = featT[4:5, 0:5]
    tf = _lnorm(jax.nn.relu(_dot(traffic, tr_wt[...]) + tr_b[...]),
                tr_g[...], tr_beta[...])

    comb = jnp.concatenate([pooled, tf], axis=1)
    out = _lnorm(jax.nn.relu(_dot(comb, fus_wt[...]) + fus_b[...]),
                 fus_g[...], fus_beta[...])
    out_ref[0] = out


def _full_spec(shape):
    nd = len(shape)
    return pl.BlockSpec(shape, lambda s, _n=nd: (0,) * _n)


def kernel(observations, params):
    p = params
    maskflat, featflat = _get_sc_extract()(observations.reshape(-1))
    mask = maskflat.reshape(_B, 2, _N, 128)
    feat = featflat.reshape(_B, 8, _N)

    heads = 4
    as_bd = (jnp.eye(heads, dtype=jnp.float32)[:, None, :]
             * p['gat1_as'][:, :, None]).reshape(heads * 64, heads)
    ad_bd = (jnp.eye(heads, dtype=jnp.float32)[:, None, :]
             * p['gat1_ad'][:, :, None]).reshape(heads * 64, heads)

    weights = [
        p['enc_W'].T,
        p['enc_b'][None, :], p['enc_g'][None, :], p['enc_beta'][None, :],
        p['proj_W'].T,
        p['proj_b'][None, :],
        p['gat1_W'].T,
        as_bd, ad_bd,
        p['gat1_b'][None, :], p['n1_g'][None, :], p['n1_b'][None, :],
        p['gat2_W'].T,
        p['gat2_as'].T, p['gat2_ad'].T,
        p['gat2_b'][None, :], p['n2_g'][None, :], p['n2_b'][None, :],
        p['tr_W'].T,
        p['tr_b'][None, :], p['tr_g'][None, :], p['tr_beta'][None, :],
        p['fus_W'].T,
        p['fus_b'][None, :], p['fus_g'][None, :], p['fus_beta'][None, :],
    ]

    in_specs = [
        pl.BlockSpec((1, 2, _N, 128), lambda s: (s, 0, 0, 0)),
        pl.BlockSpec((1, 8, _N), lambda s: (s, 0, 0)),
    ] + [_full_spec(w.shape) for w in weights]

    out = pl.pallas_call(
        _tc_gnn_kernel,
        grid=(_B,),
        in_specs=in_specs,
        out_specs=pl.BlockSpec((1, 1, _N), lambda s: (s, 0, 0)),
        out_shape=jax.ShapeDtypeStruct((_B, 1, _N), jnp.float32),
    )(mask, feat, *weights)
    return out.reshape(_B, _N)


# speedup vs baseline: 562.5254x; 1.0199x over previous
import functools

import jax
import jax.numpy as jnp
from jax import lax
from jax.experimental import pallas as pl
from jax.experimental.pallas import tpu as pltpu
from jax.experimental.pallas import tpu_sc as plsc

_B = 4
_OBS = 262144
_N = 256
_NC = 2
_NS = 16
_NW = _NC * _NS
_CHUNK = _OBS // _NS
_ROWS_W = _N // _NW
_INLEN = 2432


def _sc_extract_kernel(obs, mask_out, feat_out,
                       cbuf0, cbuf1, cntmat, shared, allcnt,
                       inb0, inb1, mb0, mb1, posbuf, trbuf, featbuf,
                       csem, isem, osem0, osem1):
    cid = lax.axis_index("c")
    sid = lax.axis_index("s")
    wid = sid * _NC + cid
    iota = lax.iota(jnp.int32, 16)
    zeros_i = jnp.zeros((16,), jnp.int32)
    ones_i = jnp.ones((16,), jnp.int32)
    zeros_f = jnp.zeros((16,), jnp.float32)

    cbufs = [cbuf0, cbuf1]
    pend = pltpu.async_copy(obs.at[pl.ds(sid * _CHUNK, _CHUNK)],
                            cbufs[0], csem)
    for s in range(_B):
        buf = cbufs[s % 2]
        pend.wait()
        if s + 1 < _B:
            pend = pltpu.async_copy(
                obs.at[pl.ds((s + 1) * _OBS + sid * _CHUNK, _CHUNK)],
                cbufs[(s + 1) % 2], csem)

        def cbody(i, acc, _buf=buf):
            for u in range(8):
                v = _buf[pl.ds(i * 128 + u * 16, 16)]
                acc = acc + jnp.where(v != 0.0, ones_i, zeros_i)
            return acc

        cntmat[pl.ds(s * 16, 16)] = lax.fori_loop(0, _CHUNK // 128,
                                                  cbody, zeros_i)

    pltpu.sync_copy(cntmat, shared.at[pl.ds(sid * _B * 16, _B * 16)])
    plsc.subcore_barrier()
    pltpu.sync_copy(shared, allcnt)

    n_list = []
    for s in range(_B):
        tot = zeros_i
        for r in range(_NS):
            tot = tot + allcnt[pl.ds((r * _B + s) * 16, 16)]
        nz = tot[0]
        for k in range(1, 16):
            nz = nz + tot[k]
        r0 = jnp.int32(0)
        for b in [512, 256, 128, 64, 32, 16, 8, 4, 2, 1]:
            t = r0 + b
            r0 = jnp.where(t * t <= nz, t, r0)
        n_list.append(jnp.minimum(jnp.maximum(r0 >> 1, 5), _N))

    base = wid * _ROWS_W
    inbufs = [inb0, inb1]
    mbufs = [mb0, mb1]

    def _in_abase(s):
        return pl.multiple_of((s * _OBS + base * n_list[s]) & (-8), 8)

    osems = [osem0, osem1]
    pend_in = pltpu.async_copy(obs.at[pl.ds(_in_abase(0), _INLEN)],
                               inbufs[0], isem)
    out_copies = [None] * _B
    for s in range(_B):
        n_s = n_list[s]
        off0 = (s * _OBS + base * n_s) - _in_abase(s)
        pend_in.wait()
        if s + 1 < _B:
            pend_in = pltpu.async_copy(
                obs.at[pl.ds(_in_abase(s + 1), _INLEN)],
                inbufs[(s + 1) % 2], isem)
        if s >= 2:
            for c in out_copies[s - 2]:
                c.wait()
        ib = inbufs[s % 2]
        mb = mbufs[s % 2]
        for k in range(_ROWS_W):
            rowvec = iota * 0 + (base + k)
            roff = off0 + k * n_s
            for j in range(16):
                col = j * 16 + iota
                vals = plsc.load_gather(ib, [roff + col])
                act = (vals != 0.0) & (col < n_s) & (rowvec < n_s)
                mb[pl.ds((j // 8) * (_ROWS_W * 128) + k * 128
                         + (j % 8) * 16, 16)] = jnp.where(act, 1.0, 0.0)
        half = _ROWS_W * 128
        out_copies[s] = []
        for h in range(2):
            mout = pl.multiple_of(
                s * _N * _N + h * (_N * 128) + base * 128, 8)
            out_copies[s].append(pltpu.async_copy(
                mb.at[pl.ds(h * half, half)],
                mask_out.at[pl.ds(mout, half)], osems[s % 2]))
    for c in out_copies[_B - 2] + out_copies[_B - 1]:
        c.wait()

    for s in range(_B):
        n_s = n_list[s]
        n_f = n_s.astype(jnp.float32)

        @pl.when(wid == s)
        def _():
            def zbody(z, carry):
                featbuf[pl.ds(z * 16, 16)] = zeros_f
                return carry

            lax.fori_loop(32, 64, zbody, 0)
            lax.fori_loop(96, 128, zbody, 0)

            p0 = n_s * n_s
            ap = pl.multiple_of((s * _OBS + p0) & (-8), 8)
            offp = (s * _OBS + p0) - ap
            pltpu.sync_copy(obs.at[pl.ds(ap, 896)], posbuf)
            for j in range(16):
                node = j * 16 + iota
                nm = node < n_s
                for f in range(3):
                    vals = plsc.load_gather(posbuf, [offp + 3 * node + f])
                    vals = jnp.where(nm, vals, 0.0)
                    featbuf[pl.ds((j // 8) * 1024 + f * 128
                                  + (j % 8) * 16, 16)] = vals

            t0 = 2 * p0 + 4 * n_s
            at0 = pl.multiple_of((s * _OBS + t0) & (-8), 8)
            offt = (s * _OBS + t0) - at0
            pltpu.sync_copy(obs.at[pl.ds(at0, 384)], trbuf)
            tvals = plsc.load_gather(trbuf, [offt + iota])
            featbuf[pl.ds(4 * 128, 16)] = jnp.where(iota < 5, tvals, 0.0)
            for j in range(16):
                node = j * 16 + iota
                vals = plsc.load_gather(trbuf, [offt + 5 + node])
                vals = jnp.where(node < n_s, vals, 0.0)
                featbuf[pl.ds((j // 8) * 1024 + 3 * 128
                              + (j % 8) * 16, 16)] = vals

            featbuf[pl.ds(5 * 128, 16)] = jnp.where(iota == 0, n_f, 0.0)
            pltpu.sync_copy(featbuf,
                            feat_out.at[pl.ds(s * _N * 8, _N * 8)])


@functools.cache
def _get_sc_extract():
    return functools.partial(
        pl.kernel,
        mesh=plsc.VectorSubcoreMesh(core_axis_name="c",
                                    subcore_axis_name="s"),
        compiler_params=pltpu.CompilerParams(needs_layout_passes=False),
        out_type=[
            jax.ShapeDtypeStruct((_B * _N * _N,), jnp.float32),
            jax.ShapeDtypeStruct((_B * _N * 8,), jnp.float32),
        ],
        scratch_types=[
            pltpu.VMEM((_CHUNK,), jnp.float32),
            pltpu.VMEM((_CHUNK,), jnp.float32),
            pltpu.VMEM((_B * 16,), jnp.int32),
            pltpu.VMEM_SHARED((_NS * _B * 16,), jnp.int32),
            pltpu.VMEM((_NS * _B * 16,), jnp.int32),
            pltpu.VMEM((_INLEN,), jnp.float32),
            pltpu.VMEM((_INLEN,), jnp.float32),
            pltpu.VMEM((_ROWS_W * _N,), jnp.float32),
            pltpu.VMEM((_ROWS_W * _N,), jnp.float32),
            pltpu.VMEM((896,), jnp.float32),
            pltpu.VMEM((384,), jnp.float32),
            pltpu.VMEM((_N * 8,), jnp.float32),
            pltpu.SemaphoreType.DMA,
            pltpu.SemaphoreType.DMA,
            pltpu.SemaphoreType.DMA,
            pltpu.SemaphoreType.DMA,
        ],
    )(_sc_extract_kernel)


def _tdot(a, b):
    return lax.dot_general(a, b, (((0,), (0,)), ((), ())),
                           preferred_element_type=jnp.float32)


def _dot(a, b):
    return lax.dot_general(a, b, (((1,), (0,)), ((), ())),
                           preferred_element_type=jnp.float32)


def _lnorm(x, g, b):
    m = jnp.mean(x, axis=-1, keepdims=True)
    v = jnp.mean((x - m) * (x - m), axis=-1, keepdims=True)
    return (x - m) * lax.rsqrt(v + 1e-5) * g + b


def _leaky(x):
    return jnp.where(x >= 0.0, x, 0.2 * x)


def _elu(x):
    return jnp.where(x > 0.0, x, jnp.exp(x) - 1.0)


def _attn(negmask, asrc_col, adst_row, h_feat):
    logit = _leaky(asrc_col + adst_row) + negmask
    mx = jnp.max(logit, axis=0, keepdims=True)
    p = jnp.exp(logit - mx)
    rsum = 1.0 / (jnp.sum(p, axis=0, keepdims=True) + 1e-16)
    return _tdot(p * rsum, h_feat)


def _tc_gnn_kernel(mask_ref, feat_ref,
                   enc_wt, enc_b, enc_g, enc_beta,
                   proj_wt, proj_b,
                   gat1_wt, as_bd, ad_bd, gat1_b, n1_g, n1_b,
                   gat2_wt, as2, ad2, gat2_b, n2_g, n2_b,
                   tr_wt, tr_b, tr_g, tr_beta,
                   fus_wt, fus_b, fus_g, fus_beta,
                   out_ref):
    mm = mask_ref[0]
    mask_f = jnp.concatenate([mm[0], mm[1]], axis=1)
    negmask = jnp.where(mask_f > 0.5, 0.0, -1e30)
    ff = feat_ref[0]
    featT = jnp.concatenate([ff[0], ff[1]], axis=1)
    n_f = feat_ref[0, 0, 5, 0]

    i4 = lax.broadcasted_iota(jnp.int32, (4, 4), 0)
    j4 = lax.broadcasted_iota(jnp.int32, (4, 4), 1)
    eye4 = jnp.where(i4 == j4, 1.0, 0.0)
    x_raw = _tdot(featT[0:4, :], eye4)
    x = _lnorm(jax.nn.relu(_dot(x_raw, enc_wt[...]) + enc_b[...]),
               enc_g[...], enc_beta[...])
    ident = _dot(x, proj_wt[...]) + proj_b[...]

    ii = lax.broadcasted_iota(jnp.int32, (_N, _N), 0)
    jj = lax.broadcasted_iota(jnp.int32, (_N, _N), 1)
    eye = jnp.where(ii == jj, 1.0, 0.0)

    h1 = _dot(x, gat1_wt[...])
    asrc4 = _dot(h1, as_bd[...])
    adst4t = _tdot(_dot(h1, ad_bd[...]), eye)
    heads = []
    for h in range(4):
        heads.append(_attn(negmask, asrc4[:, h:h + 1],
                           adst4t[h:h + 1, :],
                           h1[:, 64 * h:64 * h + 64]))
    g1 = jnp.concatenate(heads, axis=1) + gat1_b[...]
    g1 = _elu(_lnorm(g1, n1_g[...], n1_b[...]))

    h2 = _dot(g1, gat2_wt[...])
    a2s = _dot(h2, as2[...])
    a2dt = _tdot(_dot(h2, ad2[...]), eye)
    o2 = _attn(negmask, a2s, a2dt, h2) + gat2_b[...]
    o2 = _lnorm(o2, n2_g[...], n2_b[...])
    gfin = _elu(o2 + ident)

    node_col = lax.broadcasted_iota(jnp.int32, (_N, 1), 0).astype(jnp.float32)
    nm = jnp.where(node_col < n_f, 1.0, 0.0)
    pooled = _tdot(nm, gfin) / n_f

    traffic = featT[4:5, 0:5]
    tf = _lnorm(jax.nn.relu(_dot(traffic, tr_wt[...]) + tr_b[...]),
                tr_g[...], tr_beta[...])

    comb = jnp.concatenate([pooled, tf], axis=1)
    out = _lnorm(jax.nn.relu(_dot(comb, fus_wt[...]) + fus_b[...]),
                 fus_g[...], fus_beta[...])
    out_ref[0] = out


def _full_spec(shape):
    nd = len(shape)
    return pl.BlockSpec(shape, lambda s, _n=nd: (0,) * _n)


def kernel(observations, params):
    p = params
    maskflat, featflat = _get_sc_extract()(observations.reshape(-1))
    mask = maskflat.reshape(_B, 2, _N, 128)
    feat = featflat.reshape(_B, 2, 8, 128)

    heads = 4
    as_bd = (jnp.eye(heads, dtype=jnp.float32)[:, None, :]
             * p['gat1_as'][:, :, None]).reshape(heads * 64, heads)
    ad_bd = (jnp.eye(heads, dtype=jnp.float32)[:, None, :]
             * p['gat1_ad'][:, :, None]).reshape(heads * 64, heads)

    weights = [
        p['enc_W'].T,
        p['enc_b'][None, :], p['enc_g'][None, :], p['enc_beta'][None, :],
        p['proj_W'].T,
        p['proj_b'][None, :],
        p['gat1_W'].T,
        as_bd, ad_bd,
        p['gat1_b'][None, :], p['n1_g'][None, :], p['n1_b'][None, :],
        p['gat2_W'].T,
        p['gat2_as'].T, p['gat2_ad'].T,
        p['gat2_b'][None, :], p['n2_g'][None, :], p['n2_b'][None, :],
        p['tr_W'].T,
        p['tr_b'][None, :], p['tr_g'][None, :], p['tr_beta'][None, :],
        p['fus_W'].T,
        p['fus_b'][None, :], p['fus_g'][None, :], p['fus_beta'][None, :],
    ]

    in_specs = [
        pl.BlockSpec((1, 2, _N, 128), lambda s: (s, 0, 0, 0)),
        pl.BlockSpec((1, 2, 8, 128), lambda s: (s, 0, 0, 0)),
    ] + [_full_spec(w.shape) for w in weights]

    out = pl.pallas_call(
        _tc_gnn_kernel,
        grid=(_B,),
        in_specs=in_specs,
        out_specs=pl.BlockSpec((1, 1, _N), lambda s: (s, 0, 0)),
        out_shape=jax.ShapeDtypeStruct((_B, 1, _N), jnp.float32),
    )(mask, feat, *weights)
    return out.reshape(_B, _N)
